# baseline (speedup 1.0000x reference)
import functools

import jax
import jax.numpy as jnp
from jax.experimental import pallas as pl
from jax.experimental.pallas import tpu as pltpu
from jax.experimental.pallas import tpu_sc as plsc

D_MODEL = 2048


@functools.partial(
    pl.kernel,
    mesh=plsc.ScalarSubcoreMesh(axis_name="c", num_cores=1),
    out_type=jax.ShapeDtypeStruct((1, D_MODEL), jnp.float32),
    scratch_types=[
        pltpu.SMEM((1,), jnp.int32),
    ],
)
def _sc_row_gather(table_hbm, idx_hbm, out_hbm, idx_s):
    pltpu.sync_copy(idx_hbm, idx_s)
    i = idx_s[0]
    pltpu.sync_copy(table_hbm.at[pl.ds(i, 1)], out_hbm)


def kernel(preProcess, xx, state):
    out = _sc_row_gather(preProcess, xx)
    return (out[0], state)


# speedup vs baseline: 1.0008x; 1.0008x over previous
import functools

import jax
import jax.numpy as jnp
from jax.experimental import pallas as pl
from jax.experimental.pallas import tpu as pltpu
from jax.experimental.pallas import tpu_sc as plsc

D_MODEL = 2048


@functools.partial(
    pl.kernel,
    mesh=plsc.ScalarSubcoreMesh(axis_name="c", num_cores=1),
    out_type=jax.ShapeDtypeStruct((1, D_MODEL), jnp.float32),
    scratch_types=[
        pltpu.SMEM((1,), jnp.int32),
    ],
    compiler_params=pltpu.CompilerParams(skip_device_barrier=True),
)
def _sc_row_gather(table_hbm, idx_hbm, out_hbm, idx_s):
    pltpu.sync_copy(idx_hbm, idx_s)
    i = idx_s[0]
    pltpu.sync_copy(table_hbm.at[pl.ds(i, 1)], out_hbm)


def kernel(preProcess, xx, state):
    out = _sc_row_gather(preProcess, xx)
    return (out[0], state)
